# Initial kernel scaffold; baseline (speedup 1.0000x reference)
import jax
import jax.numpy as jnp
from jax.experimental import pallas as pl


def kernel(x, edge_index, batch, W1, b1, Wmu, bmu, Wls, bls):
    raise NotImplementedError("write your pallas kernel here")


# speedup vs baseline: 12.7951x; 12.7951x over previous
import functools

import jax
import jax.numpy as jnp
from jax import lax
from jax.experimental import pallas as pl
from jax.experimental.pallas import tpu as pltpu
from jax.experimental.pallas import tpu_sc as plsc

NC = 2
NS = 16
NW = NC * NS
CH = 128
DW = 16


def _mesh():
    return plsc.VectorSubcoreMesh(
        core_axis_name="c", subcore_axis_name="s",
        num_cores=NC, num_subcores=NS)


def _make_deg(nacc, nchunks):
    rpt = nacc // NS

    def body(dstp, zeros16, ones16, out0, out1, dst_v, ones_v, acc, sem):
        cid = lax.axis_index("c")
        sid = lax.axis_index("s")
        wid = cid * NS + sid
        sl = pl.ds(sid * rpt, rpt)
        pltpu.sync_copy(zeros16.at[sl], acc.at[sl])
        pltpu.sync_copy(ones16, ones_v)
        pltpu.sync_copy(dstp.at[wid], dst_v)
        plsc.subcore_barrier()

        def step(j, carry):
            pltpu.sync_copy(ones_v, acc.at[dst_v.at[j]], add=True)
            return carry

        lax.fori_loop(0, nchunks, step, 0)
        plsc.subcore_barrier()

        @pl.when(cid == 0)
        def _():
            pltpu.sync_copy(acc.at[sl], out0.at[sl])

        @pl.when(cid == 1)
        def _():
            pltpu.sync_copy(acc.at[sl], out1.at[sl])

    out = jax.ShapeDtypeStruct((nacc, DW), jnp.float32)
    return pl.kernel(
        body,
        out_type=(out, out),
        mesh=_mesh(),
        scratch_types=[
            pltpu.VMEM((nchunks, CH), jnp.int32),
            pltpu.VMEM((CH, DW), jnp.float32),
            pltpu.VMEM_SHARED((nacc, DW), jnp.float32),
            pltpu.SemaphoreType.DMA,
        ],
    )


def _make_agg(n, d, nacc, nchunks):
    rpt = nacc // NS
    kh = nchunks // 2

    def body(table, srcp, dstp, zeros, out0, out1,
             src_v, dst_v, rows0, rows1, acc, sem0, sem1):
        cid = lax.axis_index("c")
        sid = lax.axis_index("s")
        wid = cid * NS + sid
        sl = pl.ds(sid * rpt, rpt)
        pltpu.sync_copy(zeros.at[sl], acc.at[sl])
        plsc.subcore_barrier()

        def half(base):
            pltpu.sync_copy(srcp.at[wid, pl.ds(base * kh, kh)], src_v)
            pltpu.sync_copy(dstp.at[wid, pl.ds(base * kh, kh)], dst_v)
            pltpu.async_copy(table.at[src_v.at[0]], rows0, sem0)

            def step(t, carry):
                j = 2 * t
                pltpu.make_async_copy(table.at[src_v.at[j]], rows0, sem0).wait()
                pltpu.async_copy(table.at[src_v.at[j + 1]], rows1, sem1)
                pltpu.sync_copy(rows0, acc.at[dst_v.at[j]], add=True)
                pltpu.make_async_copy(
                    table.at[src_v.at[j + 1]], rows1, sem1).wait()

                @pl.when(j + 2 < kh)
                def _():
                    pltpu.async_copy(table.at[src_v.at[j + 2]], rows0, sem0)

                pltpu.sync_copy(rows1, acc.at[dst_v.at[j + 1]], add=True)
                return carry

            lax.fori_loop(0, kh // 2, step, 0)

        half(0)
        half(1)
        plsc.subcore_barrier()

        @pl.when(cid == 0)
        def _():
            pltpu.sync_copy(acc.at[sl], out0.at[sl])

        @pl.when(cid == 1)
        def _():
            pltpu.sync_copy(acc.at[sl], out1.at[sl])

    out = jax.ShapeDtypeStruct((nacc, d), jnp.float32)
    return pl.kernel(
        body,
        out_type=(out, out),
        mesh=_mesh(),
        scratch_types=[
            pltpu.VMEM((nchunks // 2, CH), jnp.int32),
            pltpu.VMEM((nchunks // 2, CH), jnp.int32),
            pltpu.VMEM((CH, d), jnp.float32),
            pltpu.VMEM((CH, d), jnp.float32),
            pltpu.VMEM_SHARED((nacc, d), jnp.float32),
            pltpu.SemaphoreType.DMA,
            pltpu.SemaphoreType.DMA,
        ],
    )


def _scale_body(d0, d1, x, dinv_o, xs_o):
    deg = d0[:, 0:1] + d1[:, 0:1] + 1.0
    dv = lax.rsqrt(deg)
    dinv_o[...] = dv
    xs_o[...] = x[...] * dv


def _l1_body(p0, p1, xs, dinv, w, b, hs_o):
    ax = dinv[...] * (p0[...] + p1[...] + xs[...])
    h = jnp.dot(ax, w[...], preferred_element_type=jnp.float32) + b[...]
    hs_o[...] = jnp.maximum(h, 0.0) * dinv[...]


def _l2_body(q0, q1, hs, dinv, wm, bm, wl, bl, mu_o, ls_o):
    ah = dinv[...] * (q0[...] + q1[...] + hs[...])
    mu_o[...] = jnp.dot(ah, wm[...], preferred_element_type=jnp.float32) + bm[...]
    ls_o[...] = jnp.dot(ah, wl[...], preferred_element_type=jnp.float32) + bl[...]


def _row_spec(br, w):
    return pl.BlockSpec((br, w), lambda i: (i, 0))


def _full_spec(shape):
    return pl.BlockSpec(shape, lambda i: tuple(0 for _ in shape))


def kernel(x, edge_index, batch, W1, b1, Wmu, bmu, Wls, bls):
    n, d = x.shape
    e = edge_index.shape[1]
    h1 = W1.shape[1]
    h2 = Wmu.shape[1]
    assert e % NW == 0 and d % 128 == 0 and h1 % 128 == 0
    ept = e // NW
    nchunks = -(-ept // CH)
    nchunks += nchunks % 2
    eptp = nchunks * CH
    nacc = -(-(n + 1) // (8 * NS)) * (8 * NS)

    src = edge_index[0].reshape(NW, ept)
    dst = edge_index[1].reshape(NW, ept)
    pad = ((0, 0), (0, eptp - ept))
    srcp = jnp.pad(src, pad, constant_values=0).reshape(NW, nchunks, CH)
    dstp = jnp.pad(dst, pad, constant_values=n).reshape(NW, nchunks, CH)
    zeros = jnp.zeros((nacc, d), jnp.float32)
    zeros16 = jnp.zeros((nacc, DW), jnp.float32)
    ones16 = jnp.ones((CH, DW), jnp.float32)

    deg0, deg1 = _make_deg(nacc, nchunks)(dstp, zeros16, ones16)

    br = 1000
    grid = (n // br,)
    dinv, xs = pl.pallas_call(
        _scale_body,
        grid=grid,
        in_specs=[_row_spec(br, DW), _row_spec(br, DW), _row_spec(br, d)],
        out_specs=[_row_spec(br, 1), _row_spec(br, d)],
        out_shape=[
            jax.ShapeDtypeStruct((n, 1), jnp.float32),
            jax.ShapeDtypeStruct((n, d), jnp.float32),
        ],
    )(deg0, deg1, x)

    agg = _make_agg(n, d, nacc, nchunks)
    p0, p1 = agg(xs, srcp, dstp, zeros)

    hs = pl.pallas_call(
        _l1_body,
        grid=grid,
        in_specs=[_row_spec(br, d), _row_spec(br, d), _row_spec(br, d),
                  _row_spec(br, 1), _full_spec((d, h1)), _full_spec((1, h1))],
        out_specs=_row_spec(br, h1),
        out_shape=jax.ShapeDtypeStruct((n, h1), jnp.float32),
    )(p0, p1, xs, dinv, W1, b1.reshape(1, h1))

    q0, q1 = agg(hs, srcp, dstp, zeros)

    mu, logstd = pl.pallas_call(
        _l2_body,
        grid=grid,
        in_specs=[_row_spec(br, h1), _row_spec(br, h1), _row_spec(br, h1),
                  _row_spec(br, 1), _full_spec((h1, h2)), _full_spec((1, h2)),
                  _full_spec((h1, h2)), _full_spec((1, h2))],
        out_specs=[_row_spec(br, h2), _row_spec(br, h2)],
        out_shape=[
            jax.ShapeDtypeStruct((n, h2), jnp.float32),
            jax.ShapeDtypeStruct((n, h2), jnp.float32),
        ],
    )(q0, q1, hs, dinv, Wmu, bmu.reshape(1, h2), Wls, bls.reshape(1, h2))

    return (mu, logstd)
